# Initial kernel scaffold; baseline (speedup 1.0000x reference)
import jax
import jax.numpy as jnp
from jax.experimental import pallas as pl


def kernel(x, table, W, b):
    raise NotImplementedError("write your pallas kernel here")


# speedup vs baseline: 10.7171x; 10.7171x over previous
import jax
import jax.numpy as jnp
from jax import lax
from jax.experimental import pallas as pl
from jax.experimental.pallas import tpu as pltpu
from jax.experimental.pallas import tpu_sc as plsc

B = 16384
L = 200
D = 50
NUM_LABELS = 20
PW = 32

NC = 2
NS = 16
NW = NC * NS
B_PER_W = B // NW
K = 8
IDX_BLK = 100
NBLK = K * L // IDX_BLK
NSTEPS = B_PER_W // K


def _project_body(t_ref, w_ref, o_ref):
  o_ref[...] = lax.dot_general(
      t_ref[...], w_ref[...], (((1,), (1,)), ((), ())),
      preferred_element_type=jnp.float32)


def _project(table, Wp):
  blk = 4000
  V = table.shape[0]
  return pl.pallas_call(
      _project_body,
      grid=(V // blk,),
      in_specs=[
          pl.BlockSpec((blk, D), lambda i: (i, 0)),
          pl.BlockSpec((PW, D), lambda i: (0, 0)),
      ],
      out_specs=pl.BlockSpec((blk, PW), lambda i: (i, 0)),
      out_shape=jax.ShapeDtypeStruct((V, PW), jnp.float32),
  )(table, Wp)


def _sc_bag_body(xr_hbm, p_hbm, out_hbm,
                 idx0, idx1, bufs0, bufs1, outs, sem0, sem1):
  wid = lax.axis_index("s") * NC + lax.axis_index("c")
  base_elem = wid * B_PER_W

  idx_bufs = (idx0, idx1)
  row_bufs = (bufs0, bufs1)
  sems = (sem0, sem1)

  def stage_and_fire(c, parity):
    row0 = (base_elem + c * K) * (L // IDX_BLK)
    for j in range(NBLK):
      pltpu.sync_copy(xr_hbm.at[row0 + j], idx_bufs[parity][j])
    for j in range(NBLK):
      pltpu.async_copy(p_hbm.at[idx_bufs[parity][j]],
                       row_bufs[parity][j], sems[parity])

  def drain(parity):
    for j in range(NBLK):
      pltpu.make_async_copy(p_hbm.at[idx_bufs[parity][j]],
                            row_bufs[parity][j], sems[parity]).wait()

  stage_and_fire(0, 0)

  def step(g, _):
    for parity in range(2):
      c = g * 2 + parity

      @pl.when(c + 1 < NSTEPS)
      def _():
        stage_and_fire(c + 1, 1 - parity)

      drain(parity)

      for e in range(K):
        buf_a = row_bufs[parity][2 * e]
        buf_b = row_bufs[parity][2 * e + 1]

        def row_body(l, accs, buf_a=buf_a, buf_b=buf_b):
          a00, a01, a10, a11 = accs
          a00 = a00 + buf_a[l, pl.ds(0, 16)]
          a10 = a10 + buf_a[l, pl.ds(16, 16)]
          a01 = a01 + buf_b[l, pl.ds(0, 16)]
          a11 = a11 + buf_b[l, pl.ds(16, 16)]
          return a00, a01, a10, a11

        zero = jnp.zeros((16,), jnp.float32)
        a00, a01, a10, a11 = lax.fori_loop(0, IDX_BLK, row_body,
                                           (zero, zero, zero, zero))
        outs[e, pl.ds(0, 16)] = a00 + a01
        outs[e, pl.ds(16, 16)] = a10 + a11

      pltpu.sync_copy(outs, out_hbm.at[pl.ds(base_elem + c * K, K)])
    return 0

  lax.fori_loop(0, NSTEPS // 2, step, 0)


def _sc_bag(xr, P):
  mesh = plsc.VectorSubcoreMesh(core_axis_name="c", subcore_axis_name="s")
  f = pl.kernel(
      _sc_bag_body,
      out_type=jax.ShapeDtypeStruct((B, PW), jnp.float32),
      mesh=mesh,
      scratch_types=[
          [pltpu.VMEM((IDX_BLK,), jnp.int32) for _ in range(NBLK)],
          [pltpu.VMEM((IDX_BLK,), jnp.int32) for _ in range(NBLK)],
          [pltpu.VMEM((IDX_BLK, PW), jnp.float32) for _ in range(NBLK)],
          [pltpu.VMEM((IDX_BLK, PW), jnp.float32) for _ in range(NBLK)],
          pltpu.VMEM((K, PW), jnp.float32),
          pltpu.SemaphoreType.DMA,
          pltpu.SemaphoreType.DMA,
      ],
      compiler_params=pltpu.CompilerParams(use_tc_tiling_on_sc=False),
  )
  return f(xr, P)


def _head_body(s_ref, b_ref, o_ref):
  logits = s_ref[...][:, :NUM_LABELS] * (1.0 / L) + b_ref[...]
  m = jnp.max(logits, axis=1, keepdims=True)
  z = logits - m
  lse = jnp.log(jnp.sum(jnp.exp(z), axis=1, keepdims=True))
  o_ref[...] = z - lse


def _head(s, b2d):
  blk = 2048
  return pl.pallas_call(
      _head_body,
      grid=(B // blk,),
      in_specs=[
          pl.BlockSpec((blk, PW), lambda i: (i, 0)),
          pl.BlockSpec((1, NUM_LABELS), lambda i: (0, 0)),
      ],
      out_specs=pl.BlockSpec((blk, NUM_LABELS), lambda i: (i, 0)),
      out_shape=jax.ShapeDtypeStruct((B, NUM_LABELS), jnp.float32),
  )(s, b2d)


@jax.jit
def kernel(x, table, W, b):
  xr = x.astype(jnp.int32).reshape(B * L // IDX_BLK, IDX_BLK)
  Wp = jnp.concatenate(
      [W, jnp.zeros((PW - NUM_LABELS, D), jnp.float32)], axis=0)
  P = _project(table, Wp)
  sums = _sc_bag(xr, P)
  return _head(sums, b.reshape(1, NUM_LABELS))


# speedup vs baseline: 22.5922x; 2.1081x over previous
import jax
import jax.numpy as jnp
from jax import lax
from jax.experimental import pallas as pl
from jax.experimental.pallas import tpu as pltpu
from jax.experimental.pallas import tpu_sc as plsc

B = 16384
L = 200
D = 50
NUM_LABELS = 20
PW = 32
ROWS_PER_128 = 128 // PW
V = 1000000

NC = 2
NS = 16
NW = NC * NS
B_PER_W = B // NW
K = 8
IDX_BLK = 100
NBLK = K * L // IDX_BLK
NSTEPS = B_PER_W // K


def _project_body(t0, t1, t2, t3, w_ref, o_ref):
  w = w_ref[...]
  for u, t in enumerate((t0, t1, t2, t3)):
    o_ref[:, 32 * u:32 * u + 32] = lax.dot_general(
        t[...], w, (((1,), (0,)), ((), ())),
        preferred_element_type=jnp.float32)


def _project(table, WpT):
  blk = 2000
  nblocks = V // ROWS_PER_128 // blk
  in_specs = [
      pl.BlockSpec((blk, D), lambda i, u=u: (i + u * nblocks, 0))
      for u in range(ROWS_PER_128)
  ] + [pl.BlockSpec((D, PW), lambda i: (0, 0))]
  return pl.pallas_call(
      _project_body,
      grid=(nblocks,),
      in_specs=in_specs,
      out_specs=pl.BlockSpec((blk, 128), lambda i: (i, 0)),
      out_shape=jax.ShapeDtypeStruct((V // ROWS_PER_128, 128), jnp.float32),
  )(table, table, table, table, WpT)


def _sc_bag_body(xr_hbm, p_hbm, out_hbm,
                 idx0, idx1, bufs0, bufs1, outs,
                 sem0, sem1, isem0, isem1):
  wid = lax.axis_index("s") * NC + lax.axis_index("c")
  base_elem = wid * B_PER_W

  idx_bufs = (idx0, idx1)
  row_bufs = (bufs0, bufs1)
  sems = (sem0, sem1)
  isems = (isem0, isem1)

  def idx_row0(c):
    return (base_elem + c * K) * (L // IDX_BLK)

  def start_idx(c, parity):
    for j in range(NBLK):
      pltpu.async_copy(xr_hbm.at[idx_row0(c) + j], idx_bufs[parity][j],
                       isems[parity])

  def wait_idx(c, parity):
    for j in range(NBLK):
      pltpu.make_async_copy(xr_hbm.at[idx_row0(c) + j],
                            idx_bufs[parity][j], isems[parity]).wait()

  def fire(parity):
    for j in range(NBLK):
      pltpu.async_copy(p_hbm.at[idx_bufs[parity][j]],
                       row_bufs[parity][j], sems[parity])

  def drain(parity):
    for j in range(NBLK):
      pltpu.make_async_copy(p_hbm.at[idx_bufs[parity][j]],
                            row_bufs[parity][j], sems[parity]).wait()

  start_idx(0, 0)
  wait_idx(0, 0)
  fire(0)
  start_idx(1, 1)

  def step(g, _):
    for parity in range(2):
      c = g * 2 + parity

      @pl.when(c + 1 < NSTEPS)
      def _():
        wait_idx(c + 1, 1 - parity)
        fire(1 - parity)

      drain(parity)

      @pl.when(c + 2 < NSTEPS)
      def _():
        start_idx(c + 2, parity)

      for e in range(K):
        buf_a = row_bufs[parity][2 * e]
        buf_b = row_bufs[parity][2 * e + 1]

        def row_body(i, accs, buf_a=buf_a, buf_b=buf_b):
          aA0, aA1, aB0, aB1, cA0, cA1, cB0, cB1 = accs
          r = i * 4
          for u in range(4):
            lo_a = buf_a[r + u, pl.ds(0, 16)]
            hi_a = buf_a[r + u, pl.ds(16, 16)]
            lo_b = buf_b[r + u, pl.ds(0, 16)]
            hi_b = buf_b[r + u, pl.ds(16, 16)]
            if u % 2 == 0:
              aA0 = aA0 + lo_a
              cA0 = cA0 + hi_a
              aB0 = aB0 + lo_b
              cB0 = cB0 + hi_b
            else:
              aA1 = aA1 + lo_a
              cA1 = cA1 + hi_a
              aB1 = aB1 + lo_b
              cB1 = cB1 + hi_b
          return aA0, aA1, aB0, aB1, cA0, cA1, cB0, cB1

        zero = jnp.zeros((16,), jnp.float32)
        aA0, aA1, aB0, aB1, cA0, cA1, cB0, cB1 = lax.fori_loop(
            0, IDX_BLK // 4, row_body, (zero,) * 8)
        outs[e, pl.ds(0, 16)] = (aA0 + aA1) + (aB0 + aB1)
        outs[e, pl.ds(16, 16)] = (cA0 + cA1) + (cB0 + cB1)

      pltpu.sync_copy(outs, out_hbm.at[pl.ds(base_elem + c * K, K)])
    return 0

  lax.fori_loop(0, NSTEPS // 2, step, 0)


def _sc_bag(xr, P):
  mesh = plsc.VectorSubcoreMesh(core_axis_name="c", subcore_axis_name="s")
  f = pl.kernel(
      _sc_bag_body,
      out_type=jax.ShapeDtypeStruct((B, PW), jnp.float32),
      mesh=mesh,
      scratch_types=[
          [pltpu.VMEM((IDX_BLK,), jnp.int32) for _ in range(NBLK)],
          [pltpu.VMEM((IDX_BLK,), jnp.int32) for _ in range(NBLK)],
          [pltpu.VMEM((IDX_BLK, PW), jnp.float32) for _ in range(NBLK)],
          [pltpu.VMEM((IDX_BLK, PW), jnp.float32) for _ in range(NBLK)],
          pltpu.VMEM((K, PW), jnp.float32),
          pltpu.SemaphoreType.DMA,
          pltpu.SemaphoreType.DMA,
          pltpu.SemaphoreType.DMA,
          pltpu.SemaphoreType.DMA,
      ],
      compiler_params=pltpu.CompilerParams(use_tc_tiling_on_sc=False),
  )
  return f(xr, P)


def _head_body(s_ref, b_ref, o_ref):
  logits = s_ref[...][:, :NUM_LABELS] * (1.0 / L) + b_ref[...]
  m = jnp.max(logits, axis=1, keepdims=True)
  z = logits - m
  lse = jnp.log(jnp.sum(jnp.exp(z), axis=1, keepdims=True))
  o_ref[...] = z - lse


def _head(s, b2d):
  blk = 2048
  return pl.pallas_call(
      _head_body,
      grid=(B // blk,),
      in_specs=[
          pl.BlockSpec((blk, PW), lambda i: (i, 0)),
          pl.BlockSpec((1, NUM_LABELS), lambda i: (0, 0)),
      ],
      out_specs=pl.BlockSpec((blk, NUM_LABELS), lambda i: (i, 0)),
      out_shape=jax.ShapeDtypeStruct((B, NUM_LABELS), jnp.float32),
  )(s, b2d)


@jax.jit
def kernel(x, table, W, b):
  stripe = V // ROWS_PER_128
  xi = x.astype(jnp.int32)
  xp = (xi % stripe) * ROWS_PER_128 + xi // stripe
  xr = xp.reshape(B * L // IDX_BLK, IDX_BLK)
  Wp = jnp.concatenate(
      [W, jnp.zeros((PW - NUM_LABELS, D), jnp.float32)], axis=0)
  P128 = _project(table, Wp.T)
  P = P128.reshape(V, PW)
  sums = _sc_bag(xr, P)
  return _head(sums, b.reshape(1, NUM_LABELS))


# speedup vs baseline: 22.8695x; 1.0123x over previous
import jax
import jax.numpy as jnp
from jax import lax
from jax.experimental import pallas as pl
from jax.experimental.pallas import tpu as pltpu
from jax.experimental.pallas import tpu_sc as plsc

B = 16384
L = 200
D = 50
DP = 56
NUM_LABELS = 20
PW = 32
ROWS_PER_128 = 128 // PW
V = 1000000
STRIPE = V // ROWS_PER_128

NC = 2
NS = 16
NW = NC * NS
B_PER_W = B // NW
K = 8
IDX_BLK = 40
NBLK = K * L // IDX_BLK
CHUNK_IDX = K * L
NSTEPS = B_PER_W // K


def _project_body(t0, t1, t2, t3, w_ref, o_ref):
  w = w_ref[...]
  for u, t in enumerate((t0, t1, t2, t3)):
    o_ref[:, 32 * u:32 * u + 32] = lax.dot_general(
        t[...], w, (((0,), (0,)), ((), ())),
        preferred_element_type=jnp.float32)


STRIPE_PAD = 250880


def _project(tTp, WpT):
  blk = 1280
  nblocks = STRIPE_PAD // blk
  in_specs = [
      pl.BlockSpec((DP, blk), lambda i, u=u: (0, i + u * nblocks))
      for u in range(ROWS_PER_128)
  ] + [pl.BlockSpec((DP, PW), lambda i: (0, 0))]
  return pl.pallas_call(
      _project_body,
      grid=(nblocks,),
      in_specs=in_specs,
      out_specs=pl.BlockSpec((blk, 128), lambda i: (i, 0)),
      out_shape=jax.ShapeDtypeStruct((STRIPE_PAD, 128), jnp.float32),
  )(tTp, tTp, tTp, tTp, WpT)


def _sc_bag_body(xr_hbm, p_hbm, out_hbm,
                 idx0, idx1, buf0, buf1, outs,
                 sem0, sem1, isem0, isem1):
  wid = lax.axis_index("s") * NC + lax.axis_index("c")
  base_elem = wid * B_PER_W

  idx_bufs = (idx0, idx1)
  row_bufs = (buf0, buf1)
  sems = (sem0, sem1)
  isems = (isem0, isem1)

  def idx_row0(c):
    return (base_elem + c * K) * (L // IDX_BLK)

  def start_idx(c, parity):
    pltpu.async_copy(xr_hbm.at[pl.ds(idx_row0(c), NBLK)],
                     idx_bufs[parity], isems[parity])

  def wait_idx(c, parity):
    pltpu.make_async_copy(xr_hbm.at[pl.ds(idx_row0(c), NBLK)],
                          idx_bufs[parity], isems[parity]).wait()

  def fire(parity):
    for j in range(NBLK):
      pltpu.async_copy(p_hbm.at[idx_bufs[parity].at[j]],
                       row_bufs[parity].at[pl.ds(j * IDX_BLK, IDX_BLK)],
                       sems[parity])

  def drain(parity):
    for j in range(NBLK):
      pltpu.make_async_copy(
          p_hbm.at[idx_bufs[parity].at[j]],
          row_bufs[parity].at[pl.ds(j * IDX_BLK, IDX_BLK)],
          sems[parity]).wait()

  start_idx(0, 0)
  wait_idx(0, 0)
  fire(0)
  start_idx(1, 1)

  def step(g, _):
    for parity in range(2):
      c = g * 2 + parity

      @pl.when(c + 1 < NSTEPS)
      def _():
        wait_idx(c + 1, 1 - parity)
        fire(1 - parity)

      drain(parity)

      @pl.when(c + 2 < NSTEPS)
      def _():
        start_idx(c + 2, parity)

      buf = row_bufs[parity]
      for e in range(K):
        def row_body(i, accs, e=e, buf=buf):
          a0, a1, a2, a3, c0, c1, c2, c3 = accs
          r = e * L + i * 8
          for u in range(8):
            lo = buf[r + u, pl.ds(0, 16)]
            hi = buf[r + u, pl.ds(16, 16)]
            if u % 4 == 0:
              a0 = a0 + lo
              c0 = c0 + hi
            elif u % 4 == 1:
              a1 = a1 + lo
              c1 = c1 + hi
            elif u % 4 == 2:
              a2 = a2 + lo
              c2 = c2 + hi
            else:
              a3 = a3 + lo
              c3 = c3 + hi
          return a0, a1, a2, a3, c0, c1, c2, c3

        zero = jnp.zeros((16,), jnp.float32)
        a0, a1, a2, a3, c0, c1, c2, c3 = lax.fori_loop(
            0, L // 8, row_body, (zero,) * 8)
        outs[e, pl.ds(0, 16)] = (a0 + a1) + (a2 + a3)
        outs[e, pl.ds(16, 16)] = (c0 + c1) + (c2 + c3)

      pltpu.sync_copy(outs, out_hbm.at[pl.ds(base_elem + c * K, K)])
    return 0

  lax.fori_loop(0, NSTEPS // 2, step, 0)


def _sc_bag(xr, P):
  mesh = plsc.VectorSubcoreMesh(core_axis_name="c", subcore_axis_name="s")
  f = pl.kernel(
      _sc_bag_body,
      out_type=jax.ShapeDtypeStruct((B, PW), jnp.float32),
      mesh=mesh,
      scratch_types=[
          pltpu.VMEM((NBLK, IDX_BLK), jnp.int32),
          pltpu.VMEM((NBLK, IDX_BLK), jnp.int32),
          pltpu.VMEM((CHUNK_IDX, PW), jnp.float32),
          pltpu.VMEM((CHUNK_IDX, PW), jnp.float32),
          pltpu.VMEM((K, PW), jnp.float32),
          pltpu.SemaphoreType.DMA,
          pltpu.SemaphoreType.DMA,
          pltpu.SemaphoreType.DMA,
          pltpu.SemaphoreType.DMA,
      ],
      compiler_params=pltpu.CompilerParams(use_tc_tiling_on_sc=False),
  )
  return f(xr, P)


def _head_body(s_ref, b_ref, o_ref):
  logits = s_ref[...][:, :NUM_LABELS] * (1.0 / L) + b_ref[...]
  m = jnp.max(logits, axis=1, keepdims=True)
  z = logits - m
  lse = jnp.log(jnp.sum(jnp.exp(z), axis=1, keepdims=True))
  o_ref[...] = z - lse


def _head(s, b2d):
  blk = 2048
  return pl.pallas_call(
      _head_body,
      grid=(B // blk,),
      in_specs=[
          pl.BlockSpec((blk, PW), lambda i: (i, 0)),
          pl.BlockSpec((1, NUM_LABELS), lambda i: (0, 0)),
      ],
      out_specs=pl.BlockSpec((blk, NUM_LABELS), lambda i: (i, 0)),
      out_shape=jax.ShapeDtypeStruct((B, NUM_LABELS), jnp.float32),
  )(s, b2d)


@jax.jit
def kernel(x, table, W, b):
  xi = x.astype(jnp.int32)
  xp = (xi % STRIPE) * ROWS_PER_128 + xi // STRIPE
  xr = xp.reshape(B * L // IDX_BLK, IDX_BLK)
  tT = table.T
  tTp = jnp.concatenate(
      [jnp.pad(tT[:, u * STRIPE:(u + 1) * STRIPE],
               ((0, DP - D), (0, STRIPE_PAD - STRIPE)))
       for u in range(ROWS_PER_128)], axis=1)
  WpT = jnp.zeros((DP, PW), jnp.float32).at[:D, :NUM_LABELS].set(W.T)
  P128 = _project(tTp, WpT)
  P = P128.reshape(ROWS_PER_128 * STRIPE_PAD, PW)
  sums = _sc_bag(xr, P)
  return _head(sums, b.reshape(1, NUM_LABELS))


# speedup vs baseline: 30.0313x; 1.3132x over previous
import jax
import jax.numpy as jnp
from jax import lax
from jax.experimental import pallas as pl
from jax.experimental.pallas import tpu as pltpu
from jax.experimental.pallas import tpu_sc as plsc

B = 16384
L = 200
D = 50
DP = 56
NUM_LABELS = 20
PW = 32
ROWS_PER_128 = 128 // PW
V = 1000000
STRIPE = V // ROWS_PER_128

NC = 2
NS = 16
NW = NC * NS
B_PER_W = B // NW
K = 8
IDX_BLK = 40
NBLK = K * L // IDX_BLK
CHUNK_IDX = K * L
NSTEPS = B_PER_W // K


def _project_body(t0, t1, t2, t3, w_ref, o_ref):
  w = w_ref[...]
  for u, t in enumerate((t0, t1, t2, t3)):
    o_ref[:, 32 * u:32 * u + 32] = lax.dot_general(
        t[...], w, (((0,), (0,)), ((), ())),
        preferred_element_type=jnp.float32)


STRIPE_PAD = 250880


def _project(tTp, WpT):
  blk = 2560
  nblocks = STRIPE_PAD // blk
  in_specs = [
      pl.BlockSpec((DP, blk), lambda i, u=u: (0, i + u * nblocks))
      for u in range(ROWS_PER_128)
  ] + [pl.BlockSpec((DP, PW), lambda i: (0, 0))]
  return pl.pallas_call(
      _project_body,
      grid=(nblocks,),
      in_specs=in_specs,
      out_specs=pl.BlockSpec((blk, 128), lambda i: (i, 0)),
      out_shape=jax.ShapeDtypeStruct((STRIPE_PAD, 128), jnp.float32),
  )(tTp, tTp, tTp, tTp, WpT)


def _sc_bag_body(xr_hbm, p_hbm, out_hbm,
                 idx0, idx1, buf0, buf1, outs,
                 sem0, sem1, isem0, isem1):
  wid = lax.axis_index("s") * NC + lax.axis_index("c")
  base_elem = wid * B_PER_W

  idx_bufs = (idx0, idx1)
  row_bufs = (buf0, buf1)
  sems = (sem0, sem1)
  isems = (isem0, isem1)

  def idx_row0(c):
    return (base_elem + c * K) * (L // IDX_BLK)

  def start_idx(c, parity):
    pltpu.async_copy(xr_hbm.at[pl.ds(idx_row0(c), NBLK)],
                     idx_bufs[parity], isems[parity])

  def wait_idx(c, parity):
    pltpu.make_async_copy(xr_hbm.at[pl.ds(idx_row0(c), NBLK)],
                          idx_bufs[parity], isems[parity]).wait()

  def fire(parity):
    for j in range(NBLK):
      pltpu.async_copy(p_hbm.at[idx_bufs[parity].at[j]],
                       row_bufs[parity].at[pl.ds(j * IDX_BLK, IDX_BLK)],
                       sems[parity])

  def drain(parity):
    for j in range(NBLK):
      pltpu.make_async_copy(
          p_hbm.at[idx_bufs[parity].at[j]],
          row_bufs[parity].at[pl.ds(j * IDX_BLK, IDX_BLK)],
          sems[parity]).wait()

  start_idx(0, 0)
  wait_idx(0, 0)
  fire(0)
  start_idx(1, 1)

  def step(g, _):
    for parity in range(2):
      c = g * 2 + parity

      @pl.when(c + 1 < NSTEPS)
      def _():
        wait_idx(c + 1, 1 - parity)
        fire(1 - parity)

      drain(parity)

      @pl.when(c + 2 < NSTEPS)
      def _():
        start_idx(c + 2, parity)

      buf = row_bufs[parity]
      for e in range(K):
        def row_body(i, accs, e=e, buf=buf):
          a0, a1, a2, a3, c0, c1, c2, c3 = accs
          r = e * L + i * 8
          for u in range(8):
            lo = buf[r + u, pl.ds(0, 16)]
            hi = buf[r + u, pl.ds(16, 16)]
            if u % 4 == 0:
              a0 = a0 + lo
              c0 = c0 + hi
            elif u % 4 == 1:
              a1 = a1 + lo
              c1 = c1 + hi
            elif u % 4 == 2:
              a2 = a2 + lo
              c2 = c2 + hi
            else:
              a3 = a3 + lo
              c3 = c3 + hi
          return a0, a1, a2, a3, c0, c1, c2, c3

        zero = jnp.zeros((16,), jnp.float32)
        a0, a1, a2, a3, c0, c1, c2, c3 = lax.fori_loop(
            0, L // 8, row_body, (zero,) * 8)
        outs[e, pl.ds(0, 16)] = (a0 + a1) + (a2 + a3)
        outs[e, pl.ds(16, 16)] = (c0 + c1) + (c2 + c3)

      pltpu.sync_copy(outs, out_hbm.at[pl.ds(base_elem + c * K, K)])
    return 0

  lax.fori_loop(0, NSTEPS // 2, step, 0)


def _sc_bag(xr, P):
  mesh = plsc.VectorSubcoreMesh(core_axis_name="c", subcore_axis_name="s")
  f = pl.kernel(
      _sc_bag_body,
      out_type=jax.ShapeDtypeStruct((B, PW), jnp.float32),
      mesh=mesh,
      scratch_types=[
          pltpu.VMEM((NBLK, IDX_BLK), jnp.int32),
          pltpu.VMEM((NBLK, IDX_BLK), jnp.int32),
          pltpu.VMEM((CHUNK_IDX, PW), jnp.float32),
          pltpu.VMEM((CHUNK_IDX, PW), jnp.float32),
          pltpu.VMEM((K, PW), jnp.float32),
          pltpu.SemaphoreType.DMA,
          pltpu.SemaphoreType.DMA,
          pltpu.SemaphoreType.DMA,
          pltpu.SemaphoreType.DMA,
      ],
      compiler_params=pltpu.CompilerParams(use_tc_tiling_on_sc=False),
  )
  return f(xr, P)


def _head_body(s_ref, b_ref, o_ref):
  logits = s_ref[...][:, :NUM_LABELS] * (1.0 / L) + b_ref[...]
  m = jnp.max(logits, axis=1, keepdims=True)
  z = logits - m
  lse = jnp.log(jnp.sum(jnp.exp(z), axis=1, keepdims=True))
  o_ref[...] = z - lse


def _head(s, b2d):
  blk = 2048
  return pl.pallas_call(
      _head_body,
      grid=(B // blk,),
      in_specs=[
          pl.BlockSpec((blk, PW), lambda i: (i, 0)),
          pl.BlockSpec((1, NUM_LABELS), lambda i: (0, 0)),
      ],
      out_specs=pl.BlockSpec((blk, NUM_LABELS), lambda i: (i, 0)),
      out_shape=jax.ShapeDtypeStruct((B, NUM_LABELS), jnp.float32),
  )(s, b2d)


@jax.jit
def kernel(x, table, W, b):
  xi = x.astype(jnp.int32)
  xp = (xi % STRIPE_PAD) * ROWS_PER_128 + xi // STRIPE_PAD
  xr = xp.reshape(B * L // IDX_BLK, IDX_BLK)
  tTp = jnp.pad(table.T,
                ((0, DP - D), (0, ROWS_PER_128 * STRIPE_PAD - V)))
  WpT = jnp.zeros((DP, PW), jnp.float32).at[:D, :NUM_LABELS].set(W.T)
  P128 = _project(tTp, WpT)
  P = P128.reshape(ROWS_PER_128 * STRIPE_PAD, PW)
  sums = _sc_bag(xr, P)
  return _head(sums, b.reshape(1, NUM_LABELS))
